# Initial kernel scaffold; baseline (speedup 1.0000x reference)
import jax
import jax.numpy as jnp
from jax.experimental import pallas as pl


def kernel(x, pron_counts):
    raise NotImplementedError("write your pallas kernel here")


# speedup vs baseline: 4.4893x; 4.4893x over previous
import functools

import jax
import jax.numpy as jnp
from jax import lax
from jax.experimental import pallas as pl
from jax.experimental.pallas import tpu as pltpu
from jax.experimental.pallas import tpu_sc as plsc

NUM_CORES = 2
NUM_SUBCORES = 16
NUM_WORKERS = NUM_CORES * NUM_SUBCORES

V, D = 1000, 64
B = 4096 * 50
IDX_MINOR = 128
NROWS = B // IDX_MINOR
ROWS_PER_W = NROWS // NUM_WORKERS


def _norm_body(counts_ref, prob_ref):
    c = counts_ref[...]
    s = jnp.sum(c, axis=1, keepdims=True)
    prob_ref[...] = c / jnp.maximum(s, 1.0)


_normalize = pl.pallas_call(
    _norm_body,
    out_shape=jax.ShapeDtypeStruct((V, D), jnp.float32),
)


def _gather_body(idx_hbm, prob_hbm, out_hbm, idx_v, rows_v, sem):
    wid = lax.axis_index("s") * NUM_CORES + lax.axis_index("c")
    base = wid * ROWS_PER_W
    pltpu.sync_copy(idx_hbm.at[wid], idx_v)

    def step(j, carry):
        pltpu.async_copy(prob_hbm.at[idx_v.at[j]], rows_v, sem).wait()
        pltpu.sync_copy(
            rows_v, out_hbm.at[pl.ds((base + j) * IDX_MINOR, IDX_MINOR)]
        )
        return carry

    lax.fori_loop(0, ROWS_PER_W, step, 0)


_gather = functools.partial(
    pl.kernel,
    mesh=plsc.VectorSubcoreMesh(core_axis_name="c", subcore_axis_name="s"),
    out_type=jax.ShapeDtypeStruct((B, D), jnp.float32),
    scratch_types=[
        pltpu.VMEM((ROWS_PER_W, IDX_MINOR), jnp.int32),
        pltpu.VMEM((IDX_MINOR, D), jnp.float32),
        pltpu.SemaphoreType.DMA,
    ],
    compiler_params=pltpu.CompilerParams(use_tc_tiling_on_sc=False),
)(_gather_body)


def kernel(x, pron_counts):
    prob = _normalize(pron_counts)
    idx = x.reshape(NUM_WORKERS, ROWS_PER_W, IDX_MINOR)
    out = _gather(idx, prob)
    return out.reshape(x.shape[0], x.shape[1], D)


# speedup vs baseline: 4.5567x; 1.0150x over previous
import functools

import jax
import jax.numpy as jnp
from jax import lax
from jax.experimental import pallas as pl
from jax.experimental.pallas import tpu as pltpu
from jax.experimental.pallas import tpu_sc as plsc

NUM_CORES = 2
NUM_SUBCORES = 16
NUM_WORKERS = NUM_CORES * NUM_SUBCORES

V, D = 1000, 64
B = 4096 * 50
IDX_MINOR = 128
NROWS = B // IDX_MINOR
ROWS_PER_W = NROWS // NUM_WORKERS


def _norm_body(counts_ref, prob_ref):
    c = counts_ref[...]
    s = jnp.sum(c, axis=1, keepdims=True)
    prob_ref[...] = c / jnp.maximum(s, 1.0)


_normalize = pl.pallas_call(
    _norm_body,
    out_shape=jax.ShapeDtypeStruct((V, D), jnp.float32),
)


NBUF = 5
NGROUP = ROWS_PER_W // NBUF


def _gather_body(idx_hbm, prob_hbm, out_hbm, idx_v, rows_v, *sems):
    gsem, wsem = sems[:NBUF], sems[NBUF:]
    wid = lax.axis_index("s") * NUM_CORES + lax.axis_index("c")
    base = wid * ROWS_PER_W
    pltpu.sync_copy(idx_hbm.at[wid], idx_v)

    def start_gather(j, b):
        pltpu.async_copy(prob_hbm.at[idx_v.at[j]], rows_v.at[b], gsem[b])

    def wait_gather(j, b):
        pltpu.make_async_copy(
            prob_hbm.at[idx_v.at[j]], rows_v.at[b], gsem[b]
        ).wait()

    def out_slice(j):
        return out_hbm.at[pl.ds((base + j) * IDX_MINOR, IDX_MINOR)]

    def start_write(j, b):
        pltpu.async_copy(rows_v.at[b], out_slice(j), wsem[b])

    def wait_write(j, b):
        pltpu.make_async_copy(rows_v.at[b], out_slice(j), wsem[b]).wait()

    for b in range(NBUF):
        start_gather(b, b)

    def turn(g, carry):
        j0 = g * NBUF
        for b in range(NBUF):
            wait_gather(j0 + b, b)
            start_write(j0 + b, b)
        for b in range(NBUF):
            wait_write(j0 + b, b)
            start_gather(j0 + NBUF + b, b)
        return carry

    lax.fori_loop(0, NGROUP - 1, turn, 0)

    j0 = (NGROUP - 1) * NBUF
    for b in range(NBUF):
        wait_gather(j0 + b, b)
        start_write(j0 + b, b)
    for b in range(NBUF):
        wait_write(j0 + b, b)


_gather = functools.partial(
    pl.kernel,
    mesh=plsc.VectorSubcoreMesh(core_axis_name="c", subcore_axis_name="s"),
    out_type=jax.ShapeDtypeStruct((B, D), jnp.float32),
    scratch_types=(
        [
            pltpu.VMEM((ROWS_PER_W, IDX_MINOR), jnp.int32),
            pltpu.VMEM((NBUF, IDX_MINOR, D), jnp.float32),
        ]
        + [pltpu.SemaphoreType.DMA] * (2 * NBUF)
    ),
    compiler_params=pltpu.CompilerParams(use_tc_tiling_on_sc=False),
)(_gather_body)


def kernel(x, pron_counts):
    prob = _normalize(pron_counts)
    idx = x.reshape(NUM_WORKERS, ROWS_PER_W, IDX_MINOR)
    out = _gather(idx, prob)
    return out.reshape(x.shape[0], x.shape[1], D)


# speedup vs baseline: 4.5638x; 1.0016x over previous
import functools

import jax
import jax.numpy as jnp
from jax import lax
from jax.experimental import pallas as pl
from jax.experimental.pallas import tpu as pltpu
from jax.experimental.pallas import tpu_sc as plsc

NUM_CORES = 2
NUM_SUBCORES = 16
NUM_WORKERS = NUM_CORES * NUM_SUBCORES

V, D = 1000, 64
A, W = 4096, 50
A_PER_W = A // NUM_WORKERS

NBUF = 8
NGROUP = A_PER_W // NBUF


def _norm_body(counts_ref, prob_ref):
    c = counts_ref[...]
    s = jnp.sum(c, axis=1, keepdims=True)
    prob_ref[...] = c / jnp.maximum(s, 1.0)


_normalize = pl.pallas_call(
    _norm_body,
    out_shape=jax.ShapeDtypeStruct((V, D), jnp.float32),
)


def _gather_body(idx_hbm, prob_hbm, out_hbm, idx_v, rows_v, *sems):
    gsem, wsem = sems[:NBUF], sems[NBUF:]
    wid = lax.axis_index("s") * NUM_CORES + lax.axis_index("c")
    base = wid * A_PER_W
    pltpu.sync_copy(idx_hbm.at[pl.ds(base, A_PER_W)], idx_v)

    def start_gather(j, b):
        pltpu.async_copy(prob_hbm.at[idx_v.at[j]], rows_v.at[b], gsem[b])

    def wait_gather(j, b):
        pltpu.make_async_copy(
            prob_hbm.at[idx_v.at[j]], rows_v.at[b], gsem[b]
        ).wait()

    def start_write(j, b):
        pltpu.async_copy(rows_v.at[b], out_hbm.at[base + j], wsem[b])

    def wait_write(j, b):
        pltpu.make_async_copy(rows_v.at[b], out_hbm.at[base + j], wsem[b]).wait()

    for b in range(NBUF):
        start_gather(b, b)

    def turn(g, carry):
        j0 = g * NBUF
        for b in range(NBUF):
            wait_gather(j0 + b, b)
            start_write(j0 + b, b)
        for b in range(NBUF):
            wait_write(j0 + b, b)
            start_gather(j0 + NBUF + b, b)
        return carry

    lax.fori_loop(0, NGROUP - 1, turn, 0)

    j0 = (NGROUP - 1) * NBUF
    for b in range(NBUF):
        wait_gather(j0 + b, b)
        start_write(j0 + b, b)
    for b in range(NBUF):
        wait_write(j0 + b, b)


_gather = functools.partial(
    pl.kernel,
    mesh=plsc.VectorSubcoreMesh(core_axis_name="c", subcore_axis_name="s"),
    out_type=jax.ShapeDtypeStruct((A, W, D), jnp.float32),
    scratch_types=(
        [
            pltpu.VMEM((A_PER_W, W), jnp.int32),
            pltpu.VMEM((NBUF, W, D), jnp.float32),
        ]
        + [pltpu.SemaphoreType.DMA] * (2 * NBUF)
    ),
    compiler_params=pltpu.CompilerParams(use_tc_tiling_on_sc=False),
)(_gather_body)


def kernel(x, pron_counts):
    prob = _normalize(pron_counts)
    return _gather(x, prob)
